# baseline (speedup 1.0000x reference)
import functools

import jax
import jax.numpy as jnp
from jax import lax
from jax.experimental import pallas as pl
from jax.experimental.pallas import tpu as pltpu
from jax.experimental.pallas import tpu_sc as plsc

N = 10000
E_RAW = 320000
E = E_RAW + N
NP = 10240
K1 = 224
C1 = 47
EW1 = K1 * C1
EP = 32 * EW1
K = 688
C2 = 30
EW2 = K * C2
EP2 = 16 * EW2
ZR = NP // 16

BM = 1024


def _mesh():
  return plsc.VectorSubcoreMesh(
      core_axis_name="c", subcore_axis_name="s", num_cores=2, num_subcores=16)


def _tc1_body(x_ref, w_ref, as_w_ref, ad_w_ref, h_ref, as_ref, ad_ref):
  h = jnp.dot(x_ref[...], w_ref[...], preferred_element_type=jnp.float32)
  h_ref[...] = h
  as_ref[...] = jnp.dot(h, as_w_ref[...], preferred_element_type=jnp.float32)
  ad_ref[...] = jnp.dot(h, ad_w_ref[...], preferred_element_type=jnp.float32)


def _tc1(x_pad, W1, As, Ad):
  n_blk = NP // BM
  return pl.pallas_call(
      _tc1_body,
      grid=(n_blk,),
      in_specs=[
          pl.BlockSpec((BM, 128), lambda i: (i, 0)),
          pl.BlockSpec((128, 128), lambda i: (0, 0)),
          pl.BlockSpec((128, 16), lambda i: (0, 0)),
          pl.BlockSpec((128, 16), lambda i: (0, 0)),
      ],
      out_specs=[
          pl.BlockSpec((BM, 128), lambda i: (i, 0)),
          pl.BlockSpec((BM, 16), lambda i: (i, 0)),
          pl.BlockSpec((BM, 16), lambda i: (i, 0)),
      ],
      out_shape=[
          jax.ShapeDtypeStruct((NP, 128), jnp.float32),
          jax.ShapeDtypeStruct((NP, 16), jnp.float32),
          jax.ShapeDtypeStruct((NP, 16), jnp.float32),
      ],
  )(x_pad, W1, As, Ad)


def _sc1_body(src_hbm, dst_hbm, as_hbm, ad_hbm, h_hbm,
              dp, pr,
              idx_s, idx_d, gs, gd, hr, dacc, macc,
              sem0, sem1, sem2):
  cid = lax.axis_index("c")
  sid = lax.axis_index("s")

  zv = jnp.zeros((16,), jnp.float32)

  def zb(k, _):
    for q in range(8):
      hr[k, pl.ds(q * 16, 16)] = zv
    gd[k, :] = zv
    return 0
  lax.fori_loop(0, 128, zb, 0, unroll=2)
  for z in range(ZR // 128):
    pltpu.sync_copy(hr.at[pl.ds(0, 128)],
                    macc.at[pl.ds(sid * ZR + z * 128, 128)])
    pltpu.sync_copy(gd.at[pl.ds(0, 128)],
                    dacc.at[pl.ds(sid * ZR + z * 128, 128)])
  plsc.subcore_barrier()

  wbase = (cid * 16 + sid) * EW1

  def chunk(c, _):
    base = wbase + c * K1
    pltpu.sync_copy(src_hbm.at[pl.ds(base, K1)], idx_s)
    pltpu.sync_copy(dst_hbm.at[pl.ds(base, K1)], idx_d)
    cph = pltpu.async_copy(h_hbm.at[idx_s], hr, sem2)
    cps = pltpu.async_copy(as_hbm.at[idx_s], gs, sem0)
    cpd = pltpu.async_copy(ad_hbm.at[idx_d], gd, sem1)
    cps.wait()
    cpd.wait()

    def vb(i, _):
      v = gs[i, :] + gd[i, :]
      v = jnp.where(v >= 0.0, v, 0.2 * v)
      gd[i, :] = jnp.exp(v)
      return 0
    lax.fori_loop(0, K1, vb, 0, unroll=4)

    pltpu.sync_copy(gd, dacc.at[idx_d], add=True)
    cph.wait()

    def mul(k, _):
      ve = gd[k, :]
      for q in range(8):
        sl = pl.ds(q * 16, 16)
        hr[k, sl] = hr[k, sl] * ve[q]
      return 0
    lax.fori_loop(0, K1, mul, 0)

    pltpu.sync_copy(hr, macc.at[idx_d], add=True)
    return 0

  lax.fori_loop(0, C1, chunk, 0)
  plsc.subcore_barrier()

  row0 = sid * ZR
  pltpu.sync_copy(dacc.at[pl.ds(row0, ZR)], dp.at[cid].at[pl.ds(row0, ZR)])
  pltpu.sync_copy(macc.at[pl.ds(row0, ZR)], pr.at[cid].at[pl.ds(row0, ZR)])


def _sc1(srcp, dstp, as1, ad1, h1):
  f32 = jnp.float32
  kern = pl.kernel(
      _sc1_body,
      out_type=[
          jax.ShapeDtypeStruct((2, NP, 16), f32),
          jax.ShapeDtypeStruct((2, NP, 128), f32),
      ],
      mesh=_mesh(),
      compiler_params=pltpu.CompilerParams(use_tc_tiling_on_sc=False,
                                           needs_layout_passes=False),
      scratch_types=[
          pltpu.VMEM((K1,), jnp.int32),
          pltpu.VMEM((K1,), jnp.int32),
          pltpu.VMEM((K1, 16), f32),
          pltpu.VMEM((K1, 16), f32),
          pltpu.VMEM((K1, 128), f32),
          pltpu.VMEM_SHARED((NP, 16), f32),
          pltpu.VMEM_SHARED((NP, 128), f32),
          pltpu.SemaphoreType.DMA,
          pltpu.SemaphoreType.DMA,
          pltpu.SemaphoreType.DMA,
      ],
  )
  return kern(srcp, dstp, as1, ad1, h1)


def _tc2_body(pr_ref, dp_ref, rep_ref, b1_ref, w2_ref,
              as_w_ref, ad_w_ref, h2_ref, as_ref, ad_ref):
  den = (dp_ref[0] + dp_ref[1])[:, :8]
  d128 = jnp.dot(den, rep_ref[...], preferred_element_type=jnp.float32)
  raw = pr_ref[0] + pr_ref[1]
  o = raw / (d128 + 1e-16) + b1_ref[...]
  a = jnp.where(o > 0.0, o, jnp.exp(o) - 1.0)
  h2 = jnp.dot(a, w2_ref[...], preferred_element_type=jnp.float32)
  h2_ref[0, :, :] = h2[:, :32]
  h2_ref[1, :, :] = h2[:, 32:]
  as_ref[...] = jnp.dot(h2, as_w_ref[...], preferred_element_type=jnp.float32)
  ad_ref[...] = jnp.dot(h2, ad_w_ref[...], preferred_element_type=jnp.float32)


def _tc2(pr, dp, rep, b1, W2, As2, Ad2):
  n_blk = NP // BM
  return pl.pallas_call(
      _tc2_body,
      grid=(n_blk,),
      in_specs=[
          pl.BlockSpec((2, BM, 128), lambda i: (0, i, 0)),
          pl.BlockSpec((2, BM, 16), lambda i: (0, i, 0)),
          pl.BlockSpec((8, 128), lambda i: (0, 0)),
          pl.BlockSpec((1, 128), lambda i: (0, 0)),
          pl.BlockSpec((128, 64), lambda i: (0, 0)),
          pl.BlockSpec((64, 8), lambda i: (0, 0)),
          pl.BlockSpec((64, 8), lambda i: (0, 0)),
      ],
      out_specs=[
          pl.BlockSpec((2, BM, 32), lambda i: (0, i, 0)),
          pl.BlockSpec((BM, 8), lambda i: (i, 0)),
          pl.BlockSpec((BM, 8), lambda i: (i, 0)),
      ],
      out_shape=[
          jax.ShapeDtypeStruct((2, NP, 32), jnp.float32),
          jax.ShapeDtypeStruct((NP, 8), jnp.float32),
          jax.ShapeDtypeStruct((NP, 8), jnp.float32),
      ],
  )(pr, dp, rep, b1, W2, As2, Ad2)


def _sc2_body(src_hbm, dst_hbm, as_hbm, ad_hbm, h_hbm,
              dq, qrlo, qrhi,
              idx_s, idx_d, ta, tb, exb, hr, dacc, macc,
              sem2):
  cid = lax.axis_index("c")
  sid = lax.axis_index("s")

  zv = jnp.zeros((16,), jnp.float32)

  def zb(k, _):
    hr[k, pl.ds(0, 16)] = zv
    hr[k, pl.ds(16, 16)] = zv
    return 0
  lax.fori_loop(0, ZR, zb, 0, unroll=2)

  def zb2(i, _):
    exb[pl.ds(i * 16, 16)] = zv
    return 0
  lax.fori_loop(0, K // 16, zb2, 0, unroll=4)
  pltpu.sync_copy(hr.at[pl.ds(0, ZR)], macc.at[pl.ds(sid * ZR, ZR)])
  pltpu.sync_copy(exb.at[pl.ds(0, ZR)], dacc.at[pl.ds(sid * ZR, ZR)])

  pltpu.sync_copy(as_hbm, ta)
  pltpu.sync_copy(ad_hbm, tb)
  plsc.subcore_barrier()

  def chunk(c, _):
    base = sid * EW2 + c * K
    pltpu.sync_copy(src_hbm.at[pl.ds(base, K)], idx_s)
    pltpu.sync_copy(dst_hbm.at[pl.ds(base, K)], idx_d)
    cph = pltpu.async_copy(h_hbm.at[cid].at[idx_s], hr, sem2)

    def vb(i, _):
      sl = pl.ds(i * 16, 16)
      a = plsc.load_gather(ta, [idx_s[sl]])
      b = plsc.load_gather(tb, [idx_d[sl]])
      v = a + b
      v = jnp.where(v >= 0.0, v, 0.2 * v)
      exb[sl] = jnp.exp(v)
      return 0
    lax.fori_loop(0, K // 16, vb, 0, unroll=4)

    @pl.when(cid == 0)
    def _():
      pltpu.sync_copy(exb, dacc.at[idx_d], add=True)
    cph.wait()

    def mul(g, _):
      ve = exb[pl.ds(g * 16, 16)]
      for j in range(16):
        s = ve[j]
        k = g * 16 + j
        hr[k, pl.ds(0, 16)] = hr[k, pl.ds(0, 16)] * s
        hr[k, pl.ds(16, 16)] = hr[k, pl.ds(16, 16)] * s
      return 0
    lax.fori_loop(0, K // 16, mul, 0)

    pltpu.sync_copy(hr, macc.at[idx_d], add=True)
    return 0

  lax.fori_loop(0, C2, chunk, 0)
  plsc.subcore_barrier()

  row0 = sid * ZR

  @pl.when(cid == 0)
  def _():
    pltpu.sync_copy(dacc.at[pl.ds(row0, ZR)], dq.at[pl.ds(row0, ZR)])
    pltpu.sync_copy(macc.at[pl.ds(row0, ZR)], qrlo.at[pl.ds(row0, ZR)])

  @pl.when(cid == 1)
  def _():
    pltpu.sync_copy(macc.at[pl.ds(row0, ZR)], qrhi.at[pl.ds(row0, ZR)])


def _sc2(srcp, dstp, as2, ad2, h2):
  f32 = jnp.float32
  kern = pl.kernel(
      _sc2_body,
      out_type=[
          jax.ShapeDtypeStruct((NP,), f32),
          jax.ShapeDtypeStruct((NP, 32), f32),
          jax.ShapeDtypeStruct((NP, 32), f32),
      ],
      mesh=_mesh(),
      compiler_params=pltpu.CompilerParams(use_tc_tiling_on_sc=False,
                                           needs_layout_passes=False),
      scratch_types=[
          pltpu.VMEM((K,), jnp.int32),
          pltpu.VMEM((K,), jnp.int32),
          pltpu.VMEM((NP,), f32),
          pltpu.VMEM((NP,), f32),
          pltpu.VMEM((K,), f32),
          pltpu.VMEM((K, 32), f32),
          pltpu.VMEM_SHARED((NP,), f32),
          pltpu.VMEM_SHARED((NP, 32), f32),
          pltpu.SemaphoreType.DMA,
      ],
  )
  return kern(srcp, dstp, as2, ad2, h2)


def _tc3_body(qr0_ref, qr1_ref, dq_ref, b2_ref, fin_ref, lsm_ref):
  den = dq_ref[...] + 1e-16
  raw = jnp.concatenate([qr0_ref[...], qr1_ref[...]], axis=1)
  o = raw / den + b2_ref[...]
  fin_ref[...] = o
  m = jnp.max(o, axis=1, keepdims=True)
  s = jnp.log(jnp.sum(jnp.exp(o - m), axis=1, keepdims=True))
  lsm_ref[...] = o - m - s


def _tc3(qr0, qr1, dq, b2):
  n_blk = NP // BM
  return pl.pallas_call(
      _tc3_body,
      grid=(n_blk,),
      in_specs=[
          pl.BlockSpec((BM, 32), lambda i: (i, 0)),
          pl.BlockSpec((BM, 32), lambda i: (i, 0)),
          pl.BlockSpec((BM, 1), lambda i: (i, 0)),
          pl.BlockSpec((1, 64), lambda i: (0, 0)),
      ],
      out_specs=[
          pl.BlockSpec((BM, 64), lambda i: (i, 0)),
          pl.BlockSpec((BM, 64), lambda i: (i, 0)),
      ],
      out_shape=[
          jax.ShapeDtypeStruct((NP, 64), jnp.float32),
          jax.ShapeDtypeStruct((NP, 64), jnp.float32),
      ],
  )(qr0, qr1, dq, b2)


def kernel(x, edge_index, edge_attr, W1, att_src1, att_dst1, b1,
           W2, att_src2, att_dst2, b2):
  del edge_attr
  f32 = jnp.float32
  i32 = jnp.int32

  loops = jnp.arange(N, dtype=i32)
  pad = jnp.full((EP - E,), N, dtype=i32)
  srcp = jnp.concatenate([edge_index[0], loops, pad])
  dstp = jnp.concatenate([edge_index[1], loops, pad])

  x_pad = jnp.pad(x, ((0, NP - N), (0, 0)))

  eye8 = jnp.eye(8, dtype=f32)
  As1 = jnp.pad((att_src1[:, :, None] * eye8[:, None, :]).reshape(128, 8),
                ((0, 0), (0, 8)))
  Ad1 = jnp.pad((att_dst1[:, :, None] * eye8[:, None, :]).reshape(128, 8),
                ((0, 0), (0, 8)))
  rep = jnp.kron(eye8, jnp.ones((1, 16), f32))
  As2 = att_src2.reshape(64, 1)
  Ad2 = att_dst2.reshape(64, 1)
  As2p = jnp.pad(As2, ((0, 0), (0, 7)))
  Ad2p = jnp.pad(Ad2, ((0, 0), (0, 7)))

  h1, as1, ad1 = _tc1(x_pad, W1, As1, Ad1)
  dp, pr = _sc1(srcp, dstp, as1, ad1, h1)
  h2, as2m, ad2m = _tc2(pr, dp, rep, b1.reshape(1, 128), W2, As2p, Ad2p)
  as2 = as2m[:, 0]
  ad2 = ad2m[:, 0]
  dq, qrlo, qrhi = _sc2(srcp, dstp, as2, ad2, h2)
  fin, lsm = _tc3(qrlo, qrhi, dq.reshape(NP, 1), b2.reshape(1, 64))
  return (fin[:N], lsm[:N])


# speedup vs baseline: 1.2355x; 1.2355x over previous
import functools

import jax
import jax.numpy as jnp
from jax import lax
from jax.experimental import pallas as pl
from jax.experimental.pallas import tpu as pltpu
from jax.experimental.pallas import tpu_sc as plsc

N = 10000
E_RAW = 320000
E = E_RAW + N
NP = 10240
K = 688
C = 30
EW = K * C
EP = 16 * EW
ZR = NP // 16

BM = 1024


def _mesh():
  return plsc.VectorSubcoreMesh(
      core_axis_name="c", subcore_axis_name="s", num_cores=2, num_subcores=16)


def _tc1_body(x_ref, w_ref, as_w_ref, ad_w_ref, h_ref, as_ref, ad_ref):
  h = jnp.dot(x_ref[...], w_ref[...], preferred_element_type=jnp.float32)
  h_ref[0, :, :] = h[:, :64]
  h_ref[1, :, :] = h[:, 64:]
  col = lax.broadcasted_iota(jnp.int32, (BM, 16), 1)
  scv = jnp.where(col < 8, 1.0, 0.2).astype(jnp.float32)
  a_s = jnp.dot(h, as_w_ref[...], preferred_element_type=jnp.float32)
  a_d = jnp.dot(h, ad_w_ref[...], preferred_element_type=jnp.float32)
  as_ref[...] = jnp.exp(a_s * scv)
  ad_ref[...] = jnp.exp(a_d * scv)


def _tc1(x_pad, W1, As, Ad):
  n_blk = NP // BM
  return pl.pallas_call(
      _tc1_body,
      grid=(n_blk,),
      in_specs=[
          pl.BlockSpec((BM, 128), lambda i: (i, 0)),
          pl.BlockSpec((128, 128), lambda i: (0, 0)),
          pl.BlockSpec((128, 16), lambda i: (0, 0)),
          pl.BlockSpec((128, 16), lambda i: (0, 0)),
      ],
      out_specs=[
          pl.BlockSpec((2, BM, 64), lambda i: (0, i, 0)),
          pl.BlockSpec((BM, 16), lambda i: (i, 0)),
          pl.BlockSpec((BM, 16), lambda i: (i, 0)),
      ],
      out_shape=[
          jax.ShapeDtypeStruct((2, NP, 64), jnp.float32),
          jax.ShapeDtypeStruct((NP, 16), jnp.float32),
          jax.ShapeDtypeStruct((NP, 16), jnp.float32),
      ],
  )(x_pad, W1, As, Ad)


def _sc1_body(src_hbm, dst_hbm, as_hbm, ad_hbm, h_hbm,
              dp, prlo, prhi,
              idx_s, idx_d, gs, gd, exb, hr, dacc, macc,
              sem0, sem1, sem2):
  cid = lax.axis_index("c")
  sid = lax.axis_index("s")

  zv = jnp.zeros((16,), jnp.float32)

  def zb(k, _):
    for q in range(4):
      hr[k, pl.ds(q * 16, 16)] = zv
    return 0
  lax.fori_loop(0, ZR, zb, 0, unroll=2)

  def zb2(i, _):
    exb[i, :] = zv
    return 0
  lax.fori_loop(0, K, zb2, 0, unroll=4)
  pltpu.sync_copy(hr.at[pl.ds(0, ZR)], macc.at[pl.ds(sid * ZR, ZR)])
  pltpu.sync_copy(exb.at[pl.ds(0, ZR)], dacc.at[pl.ds(sid * ZR, ZR)])
  plsc.subcore_barrier()

  io = lax.iota(jnp.int32, 16)
  rot = jnp.where(io < 8, io + 8, io - 8)

  def chunk(c, _):
    base = sid * EW + c * K
    pltpu.sync_copy(src_hbm.at[pl.ds(base, K)], idx_s)
    pltpu.sync_copy(dst_hbm.at[pl.ds(base, K)], idx_d)
    cph = pltpu.async_copy(h_hbm.at[cid].at[idx_s], hr, sem2)
    cps = pltpu.async_copy(as_hbm.at[idx_s], gs, sem0)
    cpd = pltpu.async_copy(ad_hbm.at[idx_d], gd, sem1)
    cps.wait()
    cpd.wait()

    def vb(i, _):
      v = gs[i, :] * gd[i, :]
      r = lax.gather(v, rot[:, None],
                     dimension_numbers=lax.GatherDimensionNumbers(
                         offset_dims=(), collapsed_slice_dims=(0,),
                         start_index_map=(0,)),
                     slice_sizes=(1,),
                     mode=lax.GatherScatterMode.PROMISE_IN_BOUNDS)
      exb[i, :] = jnp.maximum(v, r)
      return 0
    lax.fori_loop(0, K, vb, 0, unroll=4)

    @pl.when(cid == 0)
    def _():
      pltpu.sync_copy(exb, dacc.at[idx_d], add=True)
    cph.wait()

    @pl.when(cid == 0)
    def _():
      def mul0(k, _):
        ve = exb[k, :]
        for q in range(4):
          sl = pl.ds(q * 16, 16)
          hr[k, sl] = hr[k, sl] * ve[q]
        return 0
      lax.fori_loop(0, K, mul0, 0)

    @pl.when(cid == 1)
    def _():
      def mul1(k, _):
        ve = exb[k, :]
        for q in range(4):
          sl = pl.ds(q * 16, 16)
          hr[k, sl] = hr[k, sl] * ve[4 + q]
        return 0
      lax.fori_loop(0, K, mul1, 0)

    pltpu.sync_copy(hr, macc.at[idx_d], add=True)
    return 0

  lax.fori_loop(0, C, chunk, 0)
  plsc.subcore_barrier()

  row0 = sid * ZR

  @pl.when(cid == 0)
  def _():
    pltpu.sync_copy(dacc.at[pl.ds(row0, ZR)], dp.at[pl.ds(row0, ZR)])
    pltpu.sync_copy(macc.at[pl.ds(row0, ZR)], prlo.at[pl.ds(row0, ZR)])

  @pl.when(cid == 1)
  def _():
    pltpu.sync_copy(macc.at[pl.ds(row0, ZR)], prhi.at[pl.ds(row0, ZR)])


def _sc1(srcp, dstp, as1, ad1, h1):
  f32 = jnp.float32
  kern = pl.kernel(
      _sc1_body,
      out_type=[
          jax.ShapeDtypeStruct((NP, 16), f32),
          jax.ShapeDtypeStruct((NP, 64), f32),
          jax.ShapeDtypeStruct((NP, 64), f32),
      ],
      mesh=_mesh(),
      compiler_params=pltpu.CompilerParams(use_tc_tiling_on_sc=False,
                                           needs_layout_passes=False),
      scratch_types=[
          pltpu.VMEM((K,), jnp.int32),
          pltpu.VMEM((K,), jnp.int32),
          pltpu.VMEM((K, 16), f32),
          pltpu.VMEM((K, 16), f32),
          pltpu.VMEM((K, 16), f32),
          pltpu.VMEM((K, 64), f32),
          pltpu.VMEM_SHARED((NP, 16), f32),
          pltpu.VMEM_SHARED((NP, 64), f32),
          pltpu.SemaphoreType.DMA,
          pltpu.SemaphoreType.DMA,
          pltpu.SemaphoreType.DMA,
      ],
  )
  return kern(srcp, dstp, as1, ad1, h1)


def _tc2_body(pr0_ref, pr1_ref, dp_ref, rep_ref, b1_ref, w2_ref,
              as_w_ref, ad_w_ref, h2_ref, as_ref, ad_ref):
  den = dp_ref[...][:, :8]
  d128 = jnp.dot(den, rep_ref[...], preferred_element_type=jnp.float32)
  raw = jnp.concatenate([pr0_ref[...], pr1_ref[...]], axis=1)
  o = raw / (d128 + 1e-16) + b1_ref[...]
  a = jnp.where(o > 0.0, o, jnp.exp(o) - 1.0)
  h2 = jnp.dot(a, w2_ref[...], preferred_element_type=jnp.float32)
  h2_ref[0, :, :] = h2[:, :32]
  h2_ref[1, :, :] = h2[:, 32:]
  as_ref[...] = jnp.dot(h2, as_w_ref[...], preferred_element_type=jnp.float32)
  ad_ref[...] = jnp.dot(h2, ad_w_ref[...], preferred_element_type=jnp.float32)


def _tc2(pr0, pr1, dp, rep, b1, W2, As2, Ad2):
  n_blk = NP // BM
  return pl.pallas_call(
      _tc2_body,
      grid=(n_blk,),
      in_specs=[
          pl.BlockSpec((BM, 64), lambda i: (i, 0)),
          pl.BlockSpec((BM, 64), lambda i: (i, 0)),
          pl.BlockSpec((BM, 16), lambda i: (i, 0)),
          pl.BlockSpec((8, 128), lambda i: (0, 0)),
          pl.BlockSpec((1, 128), lambda i: (0, 0)),
          pl.BlockSpec((128, 64), lambda i: (0, 0)),
          pl.BlockSpec((64, 8), lambda i: (0, 0)),
          pl.BlockSpec((64, 8), lambda i: (0, 0)),
      ],
      out_specs=[
          pl.BlockSpec((2, BM, 32), lambda i: (0, i, 0)),
          pl.BlockSpec((BM, 8), lambda i: (i, 0)),
          pl.BlockSpec((BM, 8), lambda i: (i, 0)),
      ],
      out_shape=[
          jax.ShapeDtypeStruct((2, NP, 32), jnp.float32),
          jax.ShapeDtypeStruct((NP, 8), jnp.float32),
          jax.ShapeDtypeStruct((NP, 8), jnp.float32),
      ],
  )(pr0, pr1, dp, rep, b1, W2, As2, Ad2)


def _sc2_body(src_hbm, dst_hbm, as_hbm, ad_hbm, h_hbm,
              dq, qrlo, qrhi,
              idx_s, idx_d, ta, tb, exb, hr, dacc, macc,
              sem2):
  cid = lax.axis_index("c")
  sid = lax.axis_index("s")

  zv = jnp.zeros((16,), jnp.float32)

  def zb(k, _):
    hr[k, pl.ds(0, 16)] = zv
    hr[k, pl.ds(16, 16)] = zv
    return 0
  lax.fori_loop(0, ZR, zb, 0, unroll=2)

  def zb2(i, _):
    exb[pl.ds(i * 16, 16)] = zv
    return 0
  lax.fori_loop(0, K // 16, zb2, 0, unroll=4)
  pltpu.sync_copy(hr.at[pl.ds(0, ZR)], macc.at[pl.ds(sid * ZR, ZR)])
  pltpu.sync_copy(exb.at[pl.ds(0, ZR)], dacc.at[pl.ds(sid * ZR, ZR)])

  pltpu.sync_copy(as_hbm, ta)
  pltpu.sync_copy(ad_hbm, tb)
  plsc.subcore_barrier()

  def chunk(c, _):
    base = sid * EW + c * K
    pltpu.sync_copy(src_hbm.at[pl.ds(base, K)], idx_s)
    pltpu.sync_copy(dst_hbm.at[pl.ds(base, K)], idx_d)
    cph = pltpu.async_copy(h_hbm.at[cid].at[idx_s], hr, sem2)

    def vb(i, _):
      sl = pl.ds(i * 16, 16)
      a = plsc.load_gather(ta, [idx_s[sl]])
      b = plsc.load_gather(tb, [idx_d[sl]])
      v = a + b
      v = jnp.where(v >= 0.0, v, 0.2 * v)
      exb[sl] = jnp.exp(v)
      return 0
    lax.fori_loop(0, K // 16, vb, 0, unroll=4)

    @pl.when(cid == 0)
    def _():
      pltpu.sync_copy(exb, dacc.at[idx_d], add=True)
    cph.wait()

    def mul(g, _):
      ve = exb[pl.ds(g * 16, 16)]
      for j in range(16):
        s = ve[j]
        k = g * 16 + j
        hr[k, pl.ds(0, 16)] = hr[k, pl.ds(0, 16)] * s
        hr[k, pl.ds(16, 16)] = hr[k, pl.ds(16, 16)] * s
      return 0
    lax.fori_loop(0, K // 16, mul, 0)

    pltpu.sync_copy(hr, macc.at[idx_d], add=True)
    return 0

  lax.fori_loop(0, C, chunk, 0)
  plsc.subcore_barrier()

  row0 = sid * ZR

  @pl.when(cid == 0)
  def _():
    pltpu.sync_copy(dacc.at[pl.ds(row0, ZR)], dq.at[pl.ds(row0, ZR)])
    pltpu.sync_copy(macc.at[pl.ds(row0, ZR)], qrlo.at[pl.ds(row0, ZR)])

  @pl.when(cid == 1)
  def _():
    pltpu.sync_copy(macc.at[pl.ds(row0, ZR)], qrhi.at[pl.ds(row0, ZR)])


def _sc2(srcp, dstp, as2, ad2, h2):
  f32 = jnp.float32
  kern = pl.kernel(
      _sc2_body,
      out_type=[
          jax.ShapeDtypeStruct((NP,), f32),
          jax.ShapeDtypeStruct((NP, 32), f32),
          jax.ShapeDtypeStruct((NP, 32), f32),
      ],
      mesh=_mesh(),
      compiler_params=pltpu.CompilerParams(use_tc_tiling_on_sc=False,
                                           needs_layout_passes=False),
      scratch_types=[
          pltpu.VMEM((K,), jnp.int32),
          pltpu.VMEM((K,), jnp.int32),
          pltpu.VMEM((NP,), f32),
          pltpu.VMEM((NP,), f32),
          pltpu.VMEM((K,), f32),
          pltpu.VMEM((K, 32), f32),
          pltpu.VMEM_SHARED((NP,), f32),
          pltpu.VMEM_SHARED((NP, 32), f32),
          pltpu.SemaphoreType.DMA,
      ],
  )
  return kern(srcp, dstp, as2, ad2, h2)


def _tc3_body(qr0_ref, qr1_ref, dq_ref, b2_ref, fin_ref, lsm_ref):
  den = dq_ref[...] + 1e-16
  raw = jnp.concatenate([qr0_ref[...], qr1_ref[...]], axis=1)
  o = raw / den + b2_ref[...]
  fin_ref[...] = o
  m = jnp.max(o, axis=1, keepdims=True)
  s = jnp.log(jnp.sum(jnp.exp(o - m), axis=1, keepdims=True))
  lsm_ref[...] = o - m - s


def _tc3(qr0, qr1, dq, b2):
  n_blk = NP // BM
  return pl.pallas_call(
      _tc3_body,
      grid=(n_blk,),
      in_specs=[
          pl.BlockSpec((BM, 32), lambda i: (i, 0)),
          pl.BlockSpec((BM, 32), lambda i: (i, 0)),
          pl.BlockSpec((BM, 1), lambda i: (i, 0)),
          pl.BlockSpec((1, 64), lambda i: (0, 0)),
      ],
      out_specs=[
          pl.BlockSpec((BM, 64), lambda i: (i, 0)),
          pl.BlockSpec((BM, 64), lambda i: (i, 0)),
      ],
      out_shape=[
          jax.ShapeDtypeStruct((NP, 64), jnp.float32),
          jax.ShapeDtypeStruct((NP, 64), jnp.float32),
      ],
  )(qr0, qr1, dq, b2)


def kernel(x, edge_index, edge_attr, W1, att_src1, att_dst1, b1,
           W2, att_src2, att_dst2, b2):
  del edge_attr
  f32 = jnp.float32
  i32 = jnp.int32

  loops = jnp.arange(N, dtype=i32)
  pad = jnp.full((EP - E,), N, dtype=i32)
  srcp = jnp.concatenate([edge_index[0], loops, pad])
  dstp = jnp.concatenate([edge_index[1], loops, pad])

  x_pad = jnp.pad(x, ((0, NP - N), (0, 0)))

  eye8 = jnp.eye(8, dtype=f32)
  As1_8 = (att_src1[:, :, None] * eye8[:, None, :]).reshape(128, 8)
  Ad1_8 = (att_dst1[:, :, None] * eye8[:, None, :]).reshape(128, 8)
  As1 = jnp.concatenate([As1_8, As1_8], axis=1)
  Ad1 = jnp.concatenate([Ad1_8, Ad1_8], axis=1)
  rep = jnp.kron(eye8, jnp.ones((1, 16), f32))
  As2 = att_src2.reshape(64, 1)
  Ad2 = att_dst2.reshape(64, 1)
  As2p = jnp.pad(As2, ((0, 0), (0, 7)))
  Ad2p = jnp.pad(Ad2, ((0, 0), (0, 7)))

  h1, as1, ad1 = _tc1(x_pad, W1, As1, Ad1)
  dp, prlo, prhi = _sc1(srcp, dstp, as1, ad1, h1)
  h2, as2m, ad2m = _tc2(prlo, prhi, dp, rep, b1.reshape(1, 128),
                        W2, As2p, Ad2p)
  as2 = as2m[:, 0]
  ad2 = ad2m[:, 0]
  dq, qrlo, qrhi = _sc2(srcp, dstp, as2, ad2, h2)
  fin, lsm = _tc3(qrlo, qrhi, dq.reshape(NP, 1), b2.reshape(1, 64))
  return (fin[:N], lsm[:N])
